# baseline (speedup 1.0000x reference)
import functools

import numpy as np
import jax
import jax.numpy as jnp
from jax import lax
from jax.experimental import pallas as pl
from jax.experimental.pallas import tpu as pltpu
from jax.experimental.pallas import tpu_sc as plsc

N = 10000
M = 2500
E = 320000
D = 128
H = 4
DH = D // H
L = 2

NP = 10240
MNP = 12800
TW = D + 16
RSD = float(1.0 / np.sqrt(DH))

NWORK = 32
CHUNK = 48
KI = 18
NI = -(-(E + N) // (NWORK * KI * CHUNK))
NI0 = NI + 3
NI1 = 2 * NI - NI0
ET = NI * KI * CHUNK
EEP = NWORK * ET
NPAIR = KI // 2
BR = 640


def _make_sc_agg(n_out_pad):
    mesh = plsc.VectorSubcoreMesh(core_axis_name="c", subcore_axis_name="s")
    rpz = n_out_pad // 16

    @functools.partial(
        pl.kernel,
        mesh=mesh,
        compiler_params=pltpu.CompilerParams(use_tc_tiling_on_sc=False),
        out_type=jax.ShapeDtypeStruct((2, n_out_pad, TW), jnp.float32),
        scratch_types=[
            pltpu.VMEM((2, KI, CHUNK), jnp.int32),
            pltpu.VMEM((CHUNK, TW), jnp.float32),
            pltpu.VMEM((CHUNK, TW), jnp.float32),
            pltpu.VMEM_SHARED((n_out_pad, TW), jnp.float32),
            pltpu.SemaphoreType.DMA,
            pltpu.SemaphoreType.DMA,
            pltpu.SemaphoreType.DMA,
            pltpu.SemaphoreType.DMA,
        ],
    )
    def sc_agg(t_hbm, gsrc_hbm, gdst_hbm, zeros_hbm, out_hbm,
               idx_v, rows0, rows1, acc, gsem0, gsem1, ssem0, ssem1):
        c = lax.axis_index("c")
        s = lax.axis_index("s")
        pltpu.sync_copy(zeros_hbm.at[pl.ds(0, rpz)],
                        acc.at[pl.ds(s * rpz, rpz)])
        plsc.subcore_barrier()

        ni_c = jnp.where(c == 0, NI0, NI1)
        crow0 = jnp.where(c == 0, s * (NI0 * KI),
                          16 * (NI0 * KI) + s * (NI1 * KI))

        def outer(i, carry):
            blk = crow0 + i * KI
            pltpu.sync_copy(gsrc_hbm.at[pl.ds(blk, KI)], idx_v.at[0])
            pltpu.sync_copy(gdst_hbm.at[pl.ds(blk, KI)], idx_v.at[1])

            def pair(p, carry2):
                c0 = 2 * p
                c1 = 2 * p + 1

                @pl.when(p > 0)
                def _():
                    pltpu.make_async_copy(
                        rows0, acc.at[idx_v.at[1, 0]], ssem0).wait()
                    pltpu.make_async_copy(
                        rows1, acc.at[idx_v.at[1, 0]], ssem1).wait()

                g0 = pltpu.async_copy(t_hbm.at[idx_v.at[0, c0]], rows0, gsem0)
                g1 = pltpu.async_copy(t_hbm.at[idx_v.at[0, c1]], rows1, gsem1)
                g0.wait()
                pltpu.async_copy(rows0, acc.at[idx_v.at[1, c0]], ssem0,
                                 add=True)
                g1.wait()
                pltpu.async_copy(rows1, acc.at[idx_v.at[1, c1]], ssem1,
                                 add=True)
                return carry2

            lax.fori_loop(0, NPAIR, pair, 0)
            pltpu.make_async_copy(rows0, acc.at[idx_v.at[1, 0]], ssem0).wait()
            pltpu.make_async_copy(rows1, acc.at[idx_v.at[1, 0]], ssem1).wait()
            return carry

        lax.fori_loop(0, ni_c, outer, 0)
        plsc.subcore_barrier()
        pltpu.sync_copy(acc.at[pl.ds(s * rpz, rpz)],
                        out_hbm.at[c, pl.ds(s * rpz, rpz)])

    return sc_agg


def _pre_body(x_ref, wk_ref, wv_ref, sf_ref, o_ref):
    x = x_ref[...]
    k = jnp.dot(x, wk_ref[...], preferred_element_type=jnp.float32)
    v = jnp.dot(x, wv_ref[...], preferred_element_type=jnp.float32)
    p = k * sf_ref[...]
    j = lax.broadcasted_iota(jnp.int32, (D, D), 0)
    cix = lax.broadcasted_iota(jnp.int32, (D, D), 1)
    blk128 = jnp.where((j // DH) == (cix // DH), RSD, 0.0)
    u128 = jnp.exp(jnp.dot(p, blk128, preferred_element_type=jnp.float32))
    j2 = lax.broadcasted_iota(jnp.int32, (D, TW - D), 0)
    h2 = lax.broadcasted_iota(jnp.int32, (D, TW - D), 1)
    blku = jnp.where((j2 // DH) == h2, RSD, 0.0)
    uu = jnp.exp(jnp.dot(p, blku, preferred_element_type=jnp.float32))
    o_ref[...] = jnp.concatenate([v * u128, uu], axis=1)


def _pre(x_pad, wk, wv, seedf):
    n = x_pad.shape[0]
    return pl.pallas_call(
        _pre_body,
        grid=(n // BR,),
        in_specs=[
            pl.BlockSpec((BR, D), lambda i: (i, 0)),
            pl.BlockSpec((D, D), lambda i: (0, 0)),
            pl.BlockSpec((D, D), lambda i: (0, 0)),
            pl.BlockSpec((1, D), lambda i: (0, 0)),
        ],
        out_specs=pl.BlockSpec((BR, TW), lambda i: (i, 0)),
        out_shape=jax.ShapeDtypeStruct((n, TW), jnp.float32),
    )(x_pad, wk, wv, seedf)


def _normalize_ffn_ln(acc, w1, b1, w2, b2, g, bb):
    agg = acc[:, 0:D]
    den = acc[:, D:TW]
    recip = 1.0 / (den + 1e-9)
    hh = lax.broadcasted_iota(jnp.int32, (TW - D, D), 0)
    cc = lax.broadcasted_iota(jnp.int32, (TW - D, D), 1)
    expand = jnp.where(hh == (cc // DH), 1.0, 0.0)
    aggn = agg * jnp.dot(recip, expand, preferred_element_type=jnp.float32)
    ff = jnp.maximum(
        jnp.dot(aggn, w1, preferred_element_type=jnp.float32) + b1, 0.0)
    out = aggn + jnp.dot(ff, w2, preferred_element_type=jnp.float32) + b2
    mu = jnp.mean(out, axis=1, keepdims=True)
    var = jnp.mean((out - mu) ** 2, axis=1, keepdims=True)
    t = g * (out - mu) / jnp.sqrt(var + 1e-5) + bb
    return jnp.maximum(t, 0.0)


def _post_fuse_body(a_ref, b_ref, old_ref, w1_ref, b1_ref, w2_ref, b2_ref,
                    g_ref, bb_ref, fw1_ref, fw2_ref, fb_ref, o_ref):
    acc = a_ref[0] + b_ref[0]
    t = _normalize_ffn_ln(acc, w1_ref[...], b1_ref[...], w2_ref[...],
                          b2_ref[...], g_ref[...], bb_ref[...])
    o_ref[...] = (jnp.dot(old_ref[...], fw1_ref[...],
                          preferred_element_type=jnp.float32)
                  + jnp.dot(t, fw2_ref[...],
                            preferred_element_type=jnp.float32)
                  + fb_ref[...])


def _post_body(a_ref, b_ref, w1_ref, b1_ref, w2_ref, b2_ref,
               g_ref, bb_ref, o_ref):
    acc = a_ref[0] + b_ref[0]
    o_ref[...] = _normalize_ffn_ln(acc, w1_ref[...], b1_ref[...], w2_ref[...],
                                   b2_ref[...], g_ref[...], bb_ref[...])


_W_SPEC = pl.BlockSpec((D, D), lambda i: (0, 0))
_B_SPEC = pl.BlockSpec((1, D), lambda i: (0, 0))


def _post_fuse(acc, old, w1, b1, w2, b2, g, bb, fw1, fw2, fb):
    n = old.shape[0]
    return pl.pallas_call(
        _post_fuse_body,
        grid=(n // BR,),
        in_specs=[
            pl.BlockSpec((1, BR, TW), lambda i: (0, i, 0)),
            pl.BlockSpec((1, BR, TW), lambda i: (1, i, 0)),
            pl.BlockSpec((BR, D), lambda i: (i, 0)),
            _W_SPEC, _B_SPEC, _W_SPEC, _B_SPEC, _B_SPEC, _B_SPEC,
            _W_SPEC, _W_SPEC, _B_SPEC,
        ],
        out_specs=pl.BlockSpec((BR, D), lambda i: (i, 0)),
        out_shape=jax.ShapeDtypeStruct((n, D), jnp.float32),
    )(acc, acc, old, w1, b1, w2, b2, g, bb, fw1, fw2, fb)


def _post(acc, w1, b1, w2, b2, g, bb):
    n = acc.shape[1]
    return pl.pallas_call(
        _post_body,
        grid=(n // BR,),
        in_specs=[
            pl.BlockSpec((1, BR, TW), lambda i: (0, i, 0)),
            pl.BlockSpec((1, BR, TW), lambda i: (1, i, 0)),
            _W_SPEC, _B_SPEC, _W_SPEC, _B_SPEC, _B_SPEC, _B_SPEC,
        ],
        out_specs=pl.BlockSpec((BR, D), lambda i: (i, 0)),
        out_shape=jax.ShapeDtypeStruct((n, D), jnp.float32),
    )(acc, acc, w1, b1, w2, b2, g, bb)


def kernel(x_s, x_t, seed, Wk, Wv, W1, b1, W2, b2, ln_g, ln_b,
           fuse_W, fuse_b, edge_index):
    emb_s = jnp.zeros((NP, D), jnp.float32).at[:N].set(x_s)
    emb_t = (jnp.zeros((MNP, D), jnp.float32)
             .at[:M].set(x_t).at[M:M + N].set(x_s))

    self_idx = jnp.arange(N, dtype=jnp.int32)
    src_full = jnp.concatenate([edge_index[0], self_idx])
    dst_full = jnp.concatenate([edge_index[1], M + self_idx])
    padn = EEP - (E + N)
    zpad = jnp.zeros((padn,), jnp.int32)
    gsrc_v2e = jnp.concatenate([src_full, zpad]).reshape(-1, CHUNK)
    gdst_v2e = jnp.concatenate(
        [dst_full, jnp.full((padn,), MNP - 1, jnp.int32)]).reshape(-1, CHUNK)
    gsrc_e2v = jnp.concatenate([dst_full, zpad]).reshape(-1, CHUNK)
    gdst_e2v = jnp.concatenate(
        [src_full, jnp.full((padn,), NP - 1, jnp.int32)]).reshape(-1, CHUNK)
    zeros_z = jnp.zeros((MNP // 16, TW), jnp.float32)

    sc_v2e = _make_sc_agg(MNP)
    sc_e2v = _make_sc_agg(NP)

    for l in range(L):
        t1 = _pre(emb_s, Wk[l, 0], Wv[l, 0], seed[l, 0].reshape(1, D))
        acc1 = sc_v2e(t1, gsrc_v2e, gdst_v2e, zeros_z)
        emb_t = _post_fuse(acc1, emb_t,
                           W1[l, 0], b1[l, 0].reshape(1, D),
                           W2[l, 0], b2[l, 0].reshape(1, D),
                           ln_g[l, 0].reshape(1, D), ln_b[l, 0].reshape(1, D),
                           fuse_W[l, :D], fuse_W[l, D:],
                           fuse_b[l].reshape(1, D))
        t2 = _pre(emb_t, Wk[l, 1], Wv[l, 1], seed[l, 1].reshape(1, D))
        acc2 = sc_e2v(t2, gsrc_e2v, gdst_e2v, zeros_z)
        emb_s = _post(acc2,
                      W1[l, 1], b1[l, 1].reshape(1, D),
                      W2[l, 1], b2[l, 1].reshape(1, D),
                      ln_g[l, 1].reshape(1, D), ln_b[l, 1].reshape(1, D))

    return (emb_s[:N], emb_t[:M])


# speedup vs baseline: 1.0169x; 1.0169x over previous
import functools

import numpy as np
import jax
import jax.numpy as jnp
from jax import lax
from jax.experimental import pallas as pl
from jax.experimental.pallas import tpu as pltpu
from jax.experimental.pallas import tpu_sc as plsc

N = 10000
M = 2500
E = 320000
D = 128
H = 4
DH = D // H
L = 2

NP = 10240
MNP = 12800
TW = D + 16
RSD = float(1.0 / np.sqrt(DH))

NWORK = 32
CHUNK = 48
KI = 18
NI = -(-(E + N) // (NWORK * KI * CHUNK))
NI0 = NI + 2
NI1 = 2 * NI - NI0
ET = NI * KI * CHUNK
EEP = NWORK * ET
NPAIR = KI // 2
BR = 640


def _make_sc_agg(n_out_pad):
    mesh = plsc.VectorSubcoreMesh(core_axis_name="c", subcore_axis_name="s")
    rpz = n_out_pad // 16

    @functools.partial(
        pl.kernel,
        mesh=mesh,
        compiler_params=pltpu.CompilerParams(use_tc_tiling_on_sc=False),
        out_type=jax.ShapeDtypeStruct((2, n_out_pad, TW), jnp.float32),
        scratch_types=[
            pltpu.VMEM((2, KI, CHUNK), jnp.int32),
            pltpu.VMEM((CHUNK, TW), jnp.float32),
            pltpu.VMEM((CHUNK, TW), jnp.float32),
            pltpu.VMEM_SHARED((n_out_pad, TW), jnp.float32),
            pltpu.SemaphoreType.DMA,
            pltpu.SemaphoreType.DMA,
            pltpu.SemaphoreType.DMA,
            pltpu.SemaphoreType.DMA,
        ],
    )
    def sc_agg(t_hbm, gsrc_hbm, gdst_hbm, zeros_hbm, out_hbm,
               idx_v, rows0, rows1, acc, gsem0, gsem1, ssem0, ssem1):
        c = lax.axis_index("c")
        s = lax.axis_index("s")
        pltpu.sync_copy(zeros_hbm.at[pl.ds(0, rpz)],
                        acc.at[pl.ds(s * rpz, rpz)])
        plsc.subcore_barrier()

        ni_c = jnp.where(c == 0, NI0, NI1)
        crow0 = jnp.where(c == 0, s * (NI0 * KI),
                          16 * (NI0 * KI) + s * (NI1 * KI))

        def outer(i, carry):
            blk = crow0 + i * KI
            pltpu.sync_copy(gsrc_hbm.at[pl.ds(blk, KI)], idx_v.at[0])
            pltpu.sync_copy(gdst_hbm.at[pl.ds(blk, KI)], idx_v.at[1])

            def pair(p, carry2):
                c0 = 2 * p
                c1 = 2 * p + 1

                @pl.when(p > 0)
                def _():
                    pltpu.make_async_copy(
                        rows0, acc.at[idx_v.at[1, 0]], ssem0).wait()
                    pltpu.make_async_copy(
                        rows1, acc.at[idx_v.at[1, 0]], ssem1).wait()

                g0 = pltpu.async_copy(t_hbm.at[idx_v.at[0, c0]], rows0, gsem0)
                g1 = pltpu.async_copy(t_hbm.at[idx_v.at[0, c1]], rows1, gsem1)
                g0.wait()
                pltpu.async_copy(rows0, acc.at[idx_v.at[1, c0]], ssem0,
                                 add=True)
                g1.wait()
                pltpu.async_copy(rows1, acc.at[idx_v.at[1, c1]], ssem1,
                                 add=True)
                return carry2

            lax.fori_loop(0, NPAIR, pair, 0)
            pltpu.make_async_copy(rows0, acc.at[idx_v.at[1, 0]], ssem0).wait()
            pltpu.make_async_copy(rows1, acc.at[idx_v.at[1, 0]], ssem1).wait()
            return carry

        lax.fori_loop(0, ni_c, outer, 0)
        plsc.subcore_barrier()
        pltpu.sync_copy(acc.at[pl.ds(s * rpz, rpz)],
                        out_hbm.at[c, pl.ds(s * rpz, rpz)])

    return sc_agg


def _pre_body(x_ref, wk_ref, wv_ref, sf_ref, o_ref):
    x = x_ref[...]
    k = jnp.dot(x, wk_ref[...], preferred_element_type=jnp.float32)
    v = jnp.dot(x, wv_ref[...], preferred_element_type=jnp.float32)
    p = k * sf_ref[...]
    j = lax.broadcasted_iota(jnp.int32, (D, D), 0)
    cix = lax.broadcasted_iota(jnp.int32, (D, D), 1)
    blk128 = jnp.where((j // DH) == (cix // DH), RSD, 0.0)
    u128 = jnp.exp(jnp.dot(p, blk128, preferred_element_type=jnp.float32))
    j2 = lax.broadcasted_iota(jnp.int32, (D, TW - D), 0)
    h2 = lax.broadcasted_iota(jnp.int32, (D, TW - D), 1)
    blku = jnp.where((j2 // DH) == h2, RSD, 0.0)
    uu = jnp.exp(jnp.dot(p, blku, preferred_element_type=jnp.float32))
    o_ref[...] = jnp.concatenate([v * u128, uu], axis=1)


def _pre(x_pad, wk, wv, seedf):
    n = x_pad.shape[0]
    return pl.pallas_call(
        _pre_body,
        grid=(n // BR,),
        in_specs=[
            pl.BlockSpec((BR, D), lambda i: (i, 0)),
            pl.BlockSpec((D, D), lambda i: (0, 0)),
            pl.BlockSpec((D, D), lambda i: (0, 0)),
            pl.BlockSpec((1, D), lambda i: (0, 0)),
        ],
        out_specs=pl.BlockSpec((BR, TW), lambda i: (i, 0)),
        out_shape=jax.ShapeDtypeStruct((n, TW), jnp.float32),
    )(x_pad, wk, wv, seedf)


def _normalize_ffn_ln(acc, w1, b1, w2, b2, g, bb):
    agg = acc[:, 0:D]
    den = acc[:, D:TW]
    recip = 1.0 / (den + 1e-9)
    hh = lax.broadcasted_iota(jnp.int32, (TW - D, D), 0)
    cc = lax.broadcasted_iota(jnp.int32, (TW - D, D), 1)
    expand = jnp.where(hh == (cc // DH), 1.0, 0.0)
    aggn = agg * jnp.dot(recip, expand, preferred_element_type=jnp.float32)
    ff = jnp.maximum(
        jnp.dot(aggn, w1, preferred_element_type=jnp.float32) + b1, 0.0)
    out = aggn + jnp.dot(ff, w2, preferred_element_type=jnp.float32) + b2
    mu = jnp.mean(out, axis=1, keepdims=True)
    var = jnp.mean((out - mu) ** 2, axis=1, keepdims=True)
    t = g * (out - mu) / jnp.sqrt(var + 1e-5) + bb
    return jnp.maximum(t, 0.0)


def _post_fuse_body(a_ref, b_ref, old_ref, w1_ref, b1_ref, w2_ref, b2_ref,
                    g_ref, bb_ref, fw1_ref, fw2_ref, fb_ref, o_ref):
    acc = a_ref[0] + b_ref[0]
    t = _normalize_ffn_ln(acc, w1_ref[...], b1_ref[...], w2_ref[...],
                          b2_ref[...], g_ref[...], bb_ref[...])
    o_ref[...] = (jnp.dot(old_ref[...], fw1_ref[...],
                          preferred_element_type=jnp.float32)
                  + jnp.dot(t, fw2_ref[...],
                            preferred_element_type=jnp.float32)
                  + fb_ref[...])


def _post_body(a_ref, b_ref, w1_ref, b1_ref, w2_ref, b2_ref,
               g_ref, bb_ref, o_ref):
    acc = a_ref[0] + b_ref[0]
    o_ref[...] = _normalize_ffn_ln(acc, w1_ref[...], b1_ref[...], w2_ref[...],
                                   b2_ref[...], g_ref[...], bb_ref[...])


_W_SPEC = pl.BlockSpec((D, D), lambda i: (0, 0))
_B_SPEC = pl.BlockSpec((1, D), lambda i: (0, 0))


def _post_fuse(acc, old, w1, b1, w2, b2, g, bb, fw1, fw2, fb):
    n = old.shape[0]
    return pl.pallas_call(
        _post_fuse_body,
        grid=(n // BR,),
        in_specs=[
            pl.BlockSpec((1, BR, TW), lambda i: (0, i, 0)),
            pl.BlockSpec((1, BR, TW), lambda i: (1, i, 0)),
            pl.BlockSpec((BR, D), lambda i: (i, 0)),
            _W_SPEC, _B_SPEC, _W_SPEC, _B_SPEC, _B_SPEC, _B_SPEC,
            _W_SPEC, _W_SPEC, _B_SPEC,
        ],
        out_specs=pl.BlockSpec((BR, D), lambda i: (i, 0)),
        out_shape=jax.ShapeDtypeStruct((n, D), jnp.float32),
    )(acc, acc, old, w1, b1, w2, b2, g, bb, fw1, fw2, fb)


def _post(acc, w1, b1, w2, b2, g, bb):
    n = acc.shape[1]
    return pl.pallas_call(
        _post_body,
        grid=(n // BR,),
        in_specs=[
            pl.BlockSpec((1, BR, TW), lambda i: (0, i, 0)),
            pl.BlockSpec((1, BR, TW), lambda i: (1, i, 0)),
            _W_SPEC, _B_SPEC, _W_SPEC, _B_SPEC, _B_SPEC, _B_SPEC,
        ],
        out_specs=pl.BlockSpec((BR, D), lambda i: (i, 0)),
        out_shape=jax.ShapeDtypeStruct((n, D), jnp.float32),
    )(acc, acc, w1, b1, w2, b2, g, bb)


def kernel(x_s, x_t, seed, Wk, Wv, W1, b1, W2, b2, ln_g, ln_b,
           fuse_W, fuse_b, edge_index):
    emb_s = jnp.zeros((NP, D), jnp.float32).at[:N].set(x_s)
    emb_t = (jnp.zeros((MNP, D), jnp.float32)
             .at[:M].set(x_t).at[M:M + N].set(x_s))

    self_idx = jnp.arange(N, dtype=jnp.int32)
    src_full = jnp.concatenate([edge_index[0], self_idx])
    dst_full = jnp.concatenate([edge_index[1], M + self_idx])
    padn = EEP - (E + N)
    zpad = jnp.zeros((padn,), jnp.int32)
    gsrc_v2e = jnp.concatenate([src_full, zpad]).reshape(-1, CHUNK)
    gdst_v2e = jnp.concatenate(
        [dst_full, jnp.full((padn,), MNP - 1, jnp.int32)]).reshape(-1, CHUNK)
    gsrc_e2v = jnp.concatenate([dst_full, zpad]).reshape(-1, CHUNK)
    gdst_e2v = jnp.concatenate(
        [src_full, jnp.full((padn,), NP - 1, jnp.int32)]).reshape(-1, CHUNK)
    zeros_z = jnp.zeros((MNP // 16, TW), jnp.float32)

    sc_v2e = _make_sc_agg(MNP)
    sc_e2v = _make_sc_agg(NP)

    for l in range(L):
        t1 = _pre(emb_s, Wk[l, 0], Wv[l, 0], seed[l, 0].reshape(1, D))
        acc1 = sc_v2e(t1, gsrc_v2e, gdst_v2e, zeros_z)
        emb_t = _post_fuse(acc1, emb_t,
                           W1[l, 0], b1[l, 0].reshape(1, D),
                           W2[l, 0], b2[l, 0].reshape(1, D),
                           ln_g[l, 0].reshape(1, D), ln_b[l, 0].reshape(1, D),
                           fuse_W[l, :D], fuse_W[l, D:],
                           fuse_b[l].reshape(1, D))
        t2 = _pre(emb_t, Wk[l, 1], Wv[l, 1], seed[l, 1].reshape(1, D))
        acc2 = sc_e2v(t2, gsrc_e2v, gdst_e2v, zeros_z)
        emb_s = _post(acc2,
                      W1[l, 1], b1[l, 1].reshape(1, D),
                      W2[l, 1], b2[l, 1].reshape(1, D),
                      ln_g[l, 1].reshape(1, D), ln_b[l, 1].reshape(1, D))

    return (emb_s[:N], emb_t[:M])
